# baseline (speedup 1.0000x reference)
import jax
import jax.numpy as jnp
from jax import lax
from jax.experimental import pallas as pl
from jax.experimental.pallas import tpu as pltpu
from jax.experimental.pallas import tpu_sc as plsc

B = 16384
D = 32
NU = 1_000_000
NC, NS, L = 2, 16, 16
NW = NC * NS
BPW = B // NW
CH = 128
NCH = BPW // CH
NG = BPW // L
SLEN = 997376


def _mf_body(u_hbm, i_hbm, ub_hbm, uv_hbm, ib_hbm, iv_hbm, g_hbm, out_hbm,
             u_idx, i_idx, vu, vi, bu, bi, outv, gv, sem):
    wid = lax.axis_index("s") * NC + lax.axis_index("c")

    pltpu.sync_copy(u_hbm.at[wid], u_idx)
    pltpu.sync_copy(i_hbm.at[wid], i_idx)
    pltpu.sync_copy(g_hbm, gv)

    copies = []
    for c in range(NCH):
        r = pl.ds(c * CH, CH)
        copies.append(pltpu.async_copy(ub_hbm.at[u_idx.at[c]], bu.at[r], sem))
        copies.append(pltpu.async_copy(ib_hbm.at[i_idx.at[c]], bi.at[r], sem))

    for d in range(D):
        su = uv_hbm.at[d]
        si = iv_hbm.at[d]
        for c in range(NCH):
            r = pl.ds(c * CH, CH)
            copies.append(pltpu.async_copy(su.at[u_idx.at[c]], vu.at[d, r], sem))
            copies.append(pltpu.async_copy(si.at[i_idx.at[c]], vi.at[d, r], sem))
    for cp in copies:
        cp.wait()

    glob = gv[...]

    def group(gg, _):
        base = pl.multiple_of(gg * L, L)
        s = pl.ds(base, L)
        acc = bu[s] + bi[s] + glob
        for d in range(D):
            acc = acc + vu[d, s] * vi[d, s]
        outv[s] = acc
        return _

    lax.fori_loop(0, NG, group, 0)

    pltpu.sync_copy(outv, out_hbm.at[pl.ds(wid * BPW, BPW)])


@jax.jit
def _mf(u, i, user_bias, user_vec, item_bias, item_vec, glob_bias):
    mesh = plsc.VectorSubcoreMesh(core_axis_name="c", subcore_axis_name="s",
                                  num_cores=NC, num_subcores=NS)
    return pl.kernel(
        _mf_body,
        out_type=jax.ShapeDtypeStruct((B,), jnp.float32),
        mesh=mesh,
        compiler_params=pltpu.CompilerParams(
            needs_layout_passes=False, use_tc_tiling_on_sc=False),
        scratch_types=[
            pltpu.VMEM((NCH, CH), jnp.int32),
            pltpu.VMEM((NCH, CH), jnp.int32),
            pltpu.VMEM((D, BPW), jnp.float32),
            pltpu.VMEM((D, BPW), jnp.float32),
            pltpu.VMEM((BPW,), jnp.float32),
            pltpu.VMEM((BPW,), jnp.float32),
            pltpu.VMEM((BPW,), jnp.float32),
            pltpu.VMEM((L,), jnp.float32),
            pltpu.SemaphoreType.DMA,
        ],
    )(u, i, user_bias, user_vec.T, item_bias, item_vec.T, glob_bias)


def kernel(u, i, user_bias, user_vec, item_bias, item_vec, glob_bias):
    u = u.astype(jnp.int32).reshape(NW, NCH, CH)
    i = i.astype(jnp.int32).reshape(NW, NCH, CH)
    glob = jnp.broadcast_to(glob_bias.reshape(1), (L,))
    return _mf(u, i, user_bias, user_vec, item_bias, item_vec, glob)
